# baseline (speedup 1.0000x reference)
import jax
import jax.numpy as jnp
from jax import lax
from jax.experimental import pallas as pl
from jax.experimental.pallas import tpu as pltpu
from jax.experimental.pallas import tpu_sc as plsc

N_ = 10000
E_ = 160000
D_ = 256
R_ = 5
L_ = 5

NC = 2
NS = 16
NW = NC * NS

C2 = 64
NG = 200
N8 = NG * C2
GPT = 7
B = 400
K = 128
E_PAD = E_ + 2 * K
ACCR = R_ * C2 + 8
TRASHLOC = R_ * C2

assert N8 % B == 0 and N_ % B == 0


def _sc_scatter_body(hp, srcs, rloc, bi, agg,
                     bi_v, rl_v, si_v, rows_v, acc_v, sem):
    cid = lax.axis_index("c")
    sid = lax.axis_index("s")
    wid = cid * NS + sid

    pltpu.sync_copy(bi.at[pl.ds(pl.multiple_of(wid * 16, 8), 16)], bi_v)
    bivec = bi_v[...]

    zeros16 = jnp.zeros((16,), jnp.float32)

    def _zero_acc(i, _):
        for j in range(D_ // 16):
            acc_v[i, pl.ds(j * 16, 16)] = zeros16
        return 0

    for k in range(GPT):
        g = wid + NW * k
        lo = bivec[2 * k]
        hi = bivec[2 * k + 1]

        lax.fori_loop(0, ACCR, _zero_acc, 0)

        base = pl.multiple_of((lo // 8) * 8, 8)
        nch = (hi - base + K - 1) // K

        def _chunk(ci, _):
            start = pl.multiple_of(base + ci * K, 8)
            pltpu.sync_copy(srcs.at[pl.ds(start, K)], si_v)
            pltpu.sync_copy(rloc.at[pl.ds(start, K)], rl_v)
            for bb in range(K // 16):
                pos = lax.iota(jnp.int32, 16) + (start + bb * 16)
                ok = (pos >= lo) & (pos < hi)
                rl_v[pl.ds(bb * 16, 16)] = jnp.where(
                    ok, rl_v[pl.ds(bb * 16, 16)], TRASHLOC)
            pltpu.async_copy(hp.at[si_v], rows_v, sem).wait()

            def _acc16(bb, _2):
                rv = rl_v[pl.ds(bb * 16, 16)]
                ib = bb * 16
                for t in range(16):
                    row = rv[t]
                    for j in range(D_ // 16):
                        acc_v[row, pl.ds(j * 16, 16)] = (
                            acc_v[row, pl.ds(j * 16, 16)]
                            + rows_v[ib + t, pl.ds(j * 16, 16)])
                return 0

            lax.fori_loop(0, K // 16, _acc16, 0)
            return 0

        lax.fori_loop(0, nch, _chunk, 0)

        @pl.when(g < NG)
        def _wb():
            for r in range(R_):
                dst_row = pl.multiple_of(r * N8 + g * C2, 8)
                pltpu.sync_copy(acc_v.at[pl.ds(r * C2, C2)],
                                agg.at[pl.ds(dst_row, C2)])


def _sc_scatter(hp, srcs, rloc, bi):
    mesh = plsc.VectorSubcoreMesh(core_axis_name="c", subcore_axis_name="s")
    return pl.kernel(
        _sc_scatter_body,
        out_type=jax.ShapeDtypeStruct((R_ * N8, D_), jnp.float32),
        mesh=mesh,
        compiler_params=pltpu.CompilerParams(needs_layout_passes=False),
        scratch_types=[
            pltpu.VMEM((16,), jnp.int32),
            pltpu.VMEM((K,), jnp.int32),
            pltpu.VMEM((K,), jnp.int32),
            pltpu.VMEM((K, D_), jnp.float32),
            pltpu.VMEM((ACCR, D_), jnp.float32),
            pltpu.SemaphoreType.DMA,
        ],
    )(hp, srcs, rloc, bi)


def _sc_count_body(rloc, bi, cnt, bi_v, rl_v, cnt_v):
    cid = lax.axis_index("c")
    sid = lax.axis_index("s")
    wid = cid * NS + sid

    pltpu.sync_copy(bi.at[pl.ds(pl.multiple_of(wid * 16, 8), 16)], bi_v)
    bivec = bi_v[...]

    zeros16 = jnp.zeros((16,), jnp.float32)
    ones16 = jnp.ones((16,), jnp.float32)

    def _zero_cnt(i, _):
        for j in range(128 // 16):
            cnt_v[i, pl.ds(j * 16, 16)] = zeros16
        return 0

    for k in range(GPT):
        g = wid + NW * k
        lo = bivec[2 * k]
        hi = bivec[2 * k + 1]

        lax.fori_loop(0, ACCR, _zero_cnt, 0)

        base = pl.multiple_of((lo // 8) * 8, 8)
        nch = (hi - base + K - 1) // K

        def _chunk(ci, _):
            start = pl.multiple_of(base + ci * K, 8)
            pltpu.sync_copy(rloc.at[pl.ds(start, K)], rl_v)
            for bb in range(K // 16):
                pos = lax.iota(jnp.int32, 16) + (start + bb * 16)
                ok = (pos >= lo) & (pos < hi)
                rl_v[pl.ds(bb * 16, 16)] = jnp.where(
                    ok, rl_v[pl.ds(bb * 16, 16)], TRASHLOC)

            def _acc16(bb, _2):
                rv = rl_v[pl.ds(bb * 16, 16)]
                for t in range(16):
                    row = rv[t]
                    cnt_v[row, pl.ds(0, 16)] = cnt_v[row, pl.ds(0, 16)] + ones16
                return 0

            lax.fori_loop(0, K // 16, _acc16, 0)
            return 0

        lax.fori_loop(0, nch, _chunk, 0)

        @pl.when(g < NG)
        def _wb():
            for r in range(R_):
                dst_row = pl.multiple_of(r * N8 + g * C2, 8)
                pltpu.sync_copy(cnt_v.at[pl.ds(r * C2, C2)],
                                cnt.at[pl.ds(dst_row, C2)])


def _sc_count(rloc, bi):
    mesh = plsc.VectorSubcoreMesh(core_axis_name="c", subcore_axis_name="s")
    return pl.kernel(
        _sc_count_body,
        out_type=jax.ShapeDtypeStruct((R_ * N8, 128), jnp.float32),
        mesh=mesh,
        compiler_params=pltpu.CompilerParams(needs_layout_passes=False),
        scratch_types=[
            pltpu.VMEM((16,), jnp.int32),
            pltpu.VMEM((K,), jnp.int32),
            pltpu.VMEM((ACCR, 128), jnp.float32),
        ],
    )(rloc, bi)


def _tc_layer_body(hp_ref, agg_ref, cnt_ref, wroot_ref, wrel_ref, bias_ref,
                   out_ref):
    r = pl.program_id(1)

    @pl.when(r == 0)
    def _init():
        out_ref[...] = (
            jnp.dot(hp_ref[...], wroot_ref[...],
                    preferred_element_type=jnp.float32)
            + bias_ref[...])

    cnt = cnt_ref[:, 0:1]
    mean = agg_ref[...] * (1.0 / jnp.maximum(cnt, 1.0))
    out_ref[...] = out_ref[...] + jnp.dot(mean, wrel_ref[0],
                                          preferred_element_type=jnp.float32)

    @pl.when(r == R_ - 1)
    def _fin():
        out_ref[...] = jnp.maximum(out_ref[...], 0.0)


def _tc_layer(hp, agg, cnt, wroot, wrel, bias):
    nb = N_ // B
    ra = N8 // B
    return pl.pallas_call(
        _tc_layer_body,
        grid=(nb, R_),
        in_specs=[
            pl.BlockSpec((B, D_), lambda i, r: (i, 0)),
            pl.BlockSpec((B, D_), lambda i, r: (r * ra + i, 0)),
            pl.BlockSpec((B, 128), lambda i, r: (r * ra + i, 0)),
            pl.BlockSpec((D_, D_), lambda i, r: (0, 0)),
            pl.BlockSpec((1, D_, D_), lambda i, r: (r, 0, 0)),
            pl.BlockSpec((1, D_), lambda i, r: (0, 0)),
        ],
        out_specs=pl.BlockSpec((B, D_), lambda i, r: (i, 0)),
        out_shape=jax.ShapeDtypeStruct((N_, D_), jnp.float32),
    )(hp, agg, cnt, wroot, wrel, bias.reshape(1, D_))


def kernel(x, edge_index, edge_type, rel_w, root_w, bias):
    src = edge_index[0]
    dst = edge_index[1]

    order = jnp.argsort(dst)
    src_s = src[order]
    dst_s = dst[order]
    et_s = edge_type[order]
    rowloc = et_s * C2 + (dst_s % C2)
    pad = E_PAD - E_
    srcs_p = jnp.concatenate([src_s, jnp.zeros((pad,), jnp.int32)])
    rloc_p = jnp.concatenate([rowloc, jnp.full((pad,), TRASHLOC, jnp.int32)])

    big = jnp.searchsorted(dst_s, jnp.arange(NG + 1, dtype=jnp.int32) * C2
                           ).astype(jnp.int32)
    gs = (jnp.arange(NW, dtype=jnp.int32)[:, None]
          + NW * jnp.arange(GPT, dtype=jnp.int32)[None, :])
    valid = gs < NG
    gl = jnp.where(valid, big[jnp.clip(gs, 0, NG - 1)], 0)
    gh = jnp.where(valid, big[jnp.clip(gs + 1, 0, NG)], 0)
    bi = jnp.concatenate([jnp.stack([gl, gh], axis=2).reshape(NW, 2 * GPT),
                          jnp.zeros((NW, 16 - 2 * GPT), jnp.int32)],
                         axis=1).reshape(-1)

    cnt = _sc_count(rloc_p, bi)
    hp = x
    for l in range(L_):
        agg = _sc_scatter(hp, srcs_p, rloc_p, bi)
        hp = _tc_layer(hp, agg, cnt, root_w[l], rel_w[l], bias[l])
    return hp


# speedup vs baseline: 1.6689x; 1.6689x over previous
import jax
import jax.numpy as jnp
from jax import lax
from jax.experimental import pallas as pl
from jax.experimental.pallas import tpu as pltpu
from jax.experimental.pallas import tpu_sc as plsc

N_ = 10000
E_ = 160000
D_ = 256
R_ = 5
L_ = 5

NC = 2
NS = 16
NW = NC * NS

C2 = 64
NG = 200
N8 = NG * C2
GPT = 7
B = 400
K = 128
E_PAD = E_ + 2 * K
ACCR = R_ * C2 + 8
TRASHLOC = R_ * C2

assert N8 % B == 0 and N_ % B == 0


def _sc_scatter_body(hp, srcs, rloc, bi, agg,
                     bi_v, rl_v, si_v, rows_v, acc_v, sem):
    cid = lax.axis_index("c")
    sid = lax.axis_index("s")
    wid = cid * NS + sid

    pltpu.sync_copy(bi.at[pl.ds(pl.multiple_of(wid * 16, 8), 16)], bi_v)
    bivec = bi_v[...]

    zeros16 = jnp.zeros((16,), jnp.float32)

    def _zero_acc(i, _):
        for j in range(D_ // 16):
            acc_v[i, pl.ds(j * 16, 16)] = zeros16
        return 0

    for k in range(GPT):
        g = wid + NW * k
        lo = bivec[2 * k]
        hi = bivec[2 * k + 1]

        lax.fori_loop(0, ACCR, _zero_acc, 0)

        base = pl.multiple_of((lo // 8) * 8, 8)
        nch = (hi - base + K - 1) // K

        def _chunk(ci, carry):
            start = pl.multiple_of(base + ci * K, 8)
            pltpu.sync_copy(srcs.at[pl.ds(start, K)], si_v)
            pltpu.sync_copy(rloc.at[pl.ds(start, K)], rl_v)
            pltpu.async_copy(hp.at[si_v], rows_v, sem).wait()

            def _acc16(bb, c2):
                ck, acc = c2
                rv = rl_v[pl.ds(bb * 16, 16)]
                pos = lax.iota(jnp.int32, 16) + (start + bb * 16)
                ok = (pos >= lo) & (pos < hi)
                rv = jnp.where(ok, rv, TRASHLOC)
                ib = bb * 16
                for t in range(16):
                    nk = rv[t]
                    st = nk != ck
                    acc = tuple(
                        jnp.where(st, row, a + row)
                        for a, row in (
                            (acc[j], rows_v[ib + t, pl.ds(j * 16, 16)])
                            for j in range(D_ // 16)))
                    for j in range(D_ // 16):
                        acc_v[nk, pl.ds(j * 16, 16)] = acc[j]
                    ck = nk
                return (ck, acc)

            return lax.fori_loop(0, K // 16, _acc16, carry)

        carry0 = (jnp.int32(-1),
                  tuple(zeros16 for _ in range(D_ // 16)))
        lax.fori_loop(0, nch, _chunk, carry0)

        @pl.when(g < NG)
        def _wb():
            for r in range(R_):
                dst_row = pl.multiple_of(r * N8 + g * C2, 8)
                pltpu.sync_copy(acc_v.at[pl.ds(r * C2, C2)],
                                agg.at[pl.ds(dst_row, C2)])


def _sc_scatter(hp, srcs, rloc, bi):
    mesh = plsc.VectorSubcoreMesh(core_axis_name="c", subcore_axis_name="s")
    return pl.kernel(
        _sc_scatter_body,
        out_type=jax.ShapeDtypeStruct((R_ * N8, D_), jnp.float32),
        mesh=mesh,
        compiler_params=pltpu.CompilerParams(needs_layout_passes=False),
        scratch_types=[
            pltpu.VMEM((16,), jnp.int32),
            pltpu.VMEM((K,), jnp.int32),
            pltpu.VMEM((K,), jnp.int32),
            pltpu.VMEM((K, D_), jnp.float32),
            pltpu.VMEM((ACCR, D_), jnp.float32),
            pltpu.SemaphoreType.DMA,
        ],
    )(hp, srcs, rloc, bi)


def _sc_count_body(rloc, bi, cnt, bi_v, rl_v, cnt_v):
    cid = lax.axis_index("c")
    sid = lax.axis_index("s")
    wid = cid * NS + sid

    pltpu.sync_copy(bi.at[pl.ds(pl.multiple_of(wid * 16, 8), 16)], bi_v)
    bivec = bi_v[...]

    zeros16 = jnp.zeros((16,), jnp.float32)
    ones16 = jnp.ones((16,), jnp.float32)

    def _zero_cnt(i, _):
        for j in range(128 // 16):
            cnt_v[i, pl.ds(j * 16, 16)] = zeros16
        return 0

    for k in range(GPT):
        g = wid + NW * k
        lo = bivec[2 * k]
        hi = bivec[2 * k + 1]

        lax.fori_loop(0, ACCR, _zero_cnt, 0)

        base = pl.multiple_of((lo // 8) * 8, 8)
        nch = (hi - base + K - 1) // K

        def _chunk(ci, _):
            start = pl.multiple_of(base + ci * K, 8)
            pltpu.sync_copy(rloc.at[pl.ds(start, K)], rl_v)
            for bb in range(K // 16):
                pos = lax.iota(jnp.int32, 16) + (start + bb * 16)
                ok = (pos >= lo) & (pos < hi)
                rl_v[pl.ds(bb * 16, 16)] = jnp.where(
                    ok, rl_v[pl.ds(bb * 16, 16)], TRASHLOC)

            def _acc16(bb, _2):
                rv = rl_v[pl.ds(bb * 16, 16)]
                for t in range(16):
                    row = rv[t]
                    cnt_v[row, pl.ds(0, 16)] = cnt_v[row, pl.ds(0, 16)] + ones16
                return 0

            lax.fori_loop(0, K // 16, _acc16, 0)
            return 0

        lax.fori_loop(0, nch, _chunk, 0)

        @pl.when(g < NG)
        def _wb():
            for r in range(R_):
                dst_row = pl.multiple_of(r * N8 + g * C2, 8)
                pltpu.sync_copy(cnt_v.at[pl.ds(r * C2, C2)],
                                cnt.at[pl.ds(dst_row, C2)])


def _sc_count(rloc, bi):
    mesh = plsc.VectorSubcoreMesh(core_axis_name="c", subcore_axis_name="s")
    return pl.kernel(
        _sc_count_body,
        out_type=jax.ShapeDtypeStruct((R_ * N8, 128), jnp.float32),
        mesh=mesh,
        compiler_params=pltpu.CompilerParams(needs_layout_passes=False),
        scratch_types=[
            pltpu.VMEM((16,), jnp.int32),
            pltpu.VMEM((K,), jnp.int32),
            pltpu.VMEM((ACCR, 128), jnp.float32),
        ],
    )(rloc, bi)


def _tc_layer_body(hp_ref, agg_ref, cnt_ref, wroot_ref, wrel_ref, bias_ref,
                   out_ref):
    r = pl.program_id(1)

    @pl.when(r == 0)
    def _init():
        out_ref[...] = (
            jnp.dot(hp_ref[...], wroot_ref[...],
                    preferred_element_type=jnp.float32)
            + bias_ref[...])

    cnt = cnt_ref[:, 0:1]
    mean = agg_ref[...] * (1.0 / jnp.maximum(cnt, 1.0))
    out_ref[...] = out_ref[...] + jnp.dot(mean, wrel_ref[0],
                                          preferred_element_type=jnp.float32)

    @pl.when(r == R_ - 1)
    def _fin():
        out_ref[...] = jnp.maximum(out_ref[...], 0.0)


def _tc_layer(hp, agg, cnt, wroot, wrel, bias):
    nb = N_ // B
    ra = N8 // B
    return pl.pallas_call(
        _tc_layer_body,
        grid=(nb, R_),
        in_specs=[
            pl.BlockSpec((B, D_), lambda i, r: (i, 0)),
            pl.BlockSpec((B, D_), lambda i, r: (r * ra + i, 0)),
            pl.BlockSpec((B, 128), lambda i, r: (r * ra + i, 0)),
            pl.BlockSpec((D_, D_), lambda i, r: (0, 0)),
            pl.BlockSpec((1, D_, D_), lambda i, r: (r, 0, 0)),
            pl.BlockSpec((1, D_), lambda i, r: (0, 0)),
        ],
        out_specs=pl.BlockSpec((B, D_), lambda i, r: (i, 0)),
        out_shape=jax.ShapeDtypeStruct((N_, D_), jnp.float32),
    )(hp, agg, cnt, wroot, wrel, bias.reshape(1, D_))


def kernel(x, edge_index, edge_type, rel_w, root_w, bias):
    src = edge_index[0]
    dst = edge_index[1]

    order = jnp.argsort(dst * 8 + edge_type)
    src_s = src[order]
    dst_s = dst[order]
    et_s = edge_type[order]
    rowloc = et_s * C2 + (dst_s % C2)
    pad = E_PAD - E_
    srcs_p = jnp.concatenate([src_s, jnp.zeros((pad,), jnp.int32)])
    rloc_p = jnp.concatenate([rowloc, jnp.full((pad,), TRASHLOC, jnp.int32)])

    big = jnp.searchsorted(dst_s, jnp.arange(NG + 1, dtype=jnp.int32) * C2
                           ).astype(jnp.int32)
    gs = (jnp.arange(NW, dtype=jnp.int32)[:, None]
          + NW * jnp.arange(GPT, dtype=jnp.int32)[None, :])
    valid = gs < NG
    gl = jnp.where(valid, big[jnp.clip(gs, 0, NG - 1)], 0)
    gh = jnp.where(valid, big[jnp.clip(gs + 1, 0, NG)], 0)
    bi = jnp.concatenate([jnp.stack([gl, gh], axis=2).reshape(NW, 2 * GPT),
                          jnp.zeros((NW, 16 - 2 * GPT), jnp.int32)],
                         axis=1).reshape(-1)

    cnt = _sc_count(rloc_p, bi)
    hp = x
    for l in range(L_):
        agg = _sc_scatter(hp, srcs_p, rloc_p, bi)
        hp = _tc_layer(hp, agg, cnt, root_w[l], rel_w[l], bias[l])
    return hp


# speedup vs baseline: 1.6697x; 1.0005x over previous
import jax
import jax.numpy as jnp
from jax import lax
from jax.experimental import pallas as pl
from jax.experimental.pallas import tpu as pltpu
from jax.experimental.pallas import tpu_sc as plsc

N_ = 10000
E_ = 160000
D_ = 256
R_ = 5
L_ = 5

NC = 2
NS = 16
NW = NC * NS

C2 = 64
NG = 200
N8 = NG * C2
GPT = 7
B = 400
K = 128
E_PAD = E_ + 256
ACCR = R_ * C2 + 8
TRASHLOC = R_ * C2

assert N8 % B == 0 and N_ % B == 0


def _sc_scatter_body(hp, srcs, rloc, bi, agg,
                     bi_v, rl_v, si_v, rows_v, acc_v, sem):
    cid = lax.axis_index("c")
    sid = lax.axis_index("s")
    wid = cid * NS + sid

    pltpu.sync_copy(bi.at[pl.ds(pl.multiple_of(wid * 16, 8), 16)], bi_v)
    bivec = bi_v[...]

    zeros16 = jnp.zeros((16,), jnp.float32)

    def _zero_acc(i, _):
        for j in range(D_ // 16):
            acc_v[i, pl.ds(j * 16, 16)] = zeros16
        return 0

    for k in range(GPT):
        g = wid + NW * k
        lo = bivec[2 * k]
        hi = bivec[2 * k + 1]

        lax.fori_loop(0, ACCR, _zero_acc, 0)

        base = pl.multiple_of((lo // 8) * 8, 8)
        nch = (hi - base + K - 1) // K

        def _chunk(ci, carry):
            start = pl.multiple_of(base + ci * K, 8)
            pltpu.sync_copy(srcs.at[pl.ds(start, K)], si_v)
            pltpu.sync_copy(rloc.at[pl.ds(start, K)], rl_v)
            pltpu.async_copy(hp.at[si_v], rows_v, sem).wait()

            def _acc16(bb, c2):
                ck, acc = c2
                rv = rl_v[pl.ds(bb * 16, 16)]
                pos = lax.iota(jnp.int32, 16) + (start + bb * 16)
                ok = (pos >= lo) & (pos < hi)
                rv = jnp.where(ok, rv, TRASHLOC)
                ib = bb * 16
                for t in range(16):
                    nk = rv[t]
                    st = nk != ck
                    acc = tuple(
                        jnp.where(st, row, a + row)
                        for a, row in (
                            (acc[j], rows_v[ib + t, pl.ds(j * 16, 16)])
                            for j in range(D_ // 16)))
                    for j in range(D_ // 16):
                        acc_v[nk, pl.ds(j * 16, 16)] = acc[j]
                    ck = nk
                return (ck, acc)

            return lax.fori_loop(0, K // 16, _acc16, carry)

        carry0 = (jnp.int32(-1),
                  tuple(zeros16 for _ in range(D_ // 16)))
        lax.fori_loop(0, nch, _chunk, carry0)

        @pl.when(g < NG)
        def _wb():
            for r in range(R_):
                dst_row = pl.multiple_of(r * N8 + g * C2, 8)
                pltpu.sync_copy(acc_v.at[pl.ds(r * C2, C2)],
                                agg.at[pl.ds(dst_row, C2)])


def _sc_scatter(hp, srcs, rloc, bi):
    mesh = plsc.VectorSubcoreMesh(core_axis_name="c", subcore_axis_name="s")
    return pl.kernel(
        _sc_scatter_body,
        out_type=jax.ShapeDtypeStruct((R_ * N8, D_), jnp.float32),
        mesh=mesh,
        compiler_params=pltpu.CompilerParams(needs_layout_passes=False),
        scratch_types=[
            pltpu.VMEM((16,), jnp.int32),
            pltpu.VMEM((K,), jnp.int32),
            pltpu.VMEM((K,), jnp.int32),
            pltpu.VMEM((K, D_), jnp.float32),
            pltpu.VMEM((ACCR, D_), jnp.float32),
            pltpu.SemaphoreType.DMA,
        ],
    )(hp, srcs, rloc, bi)


def _sc_count_body(rloc, bi, cnt, bi_v, rl_v, cnt_v):
    cid = lax.axis_index("c")
    sid = lax.axis_index("s")
    wid = cid * NS + sid

    pltpu.sync_copy(bi.at[pl.ds(pl.multiple_of(wid * 16, 8), 16)], bi_v)
    bivec = bi_v[...]

    zeros16 = jnp.zeros((16,), jnp.float32)
    ones16 = jnp.ones((16,), jnp.float32)

    def _zero_cnt(i, _):
        cnt_v[i, pl.ds(0, 16)] = zeros16
        return 0

    for k in range(GPT):
        g = wid + NW * k
        lo = bivec[2 * k]
        hi = bivec[2 * k + 1]

        lax.fori_loop(0, ACCR, _zero_cnt, 0)

        base = pl.multiple_of((lo // 8) * 8, 8)
        nch = (hi - base + K - 1) // K

        def _chunk(ci, _):
            start = pl.multiple_of(base + ci * K, 8)
            pltpu.sync_copy(rloc.at[pl.ds(start, K)], rl_v)
            for bb in range(K // 16):
                pos = lax.iota(jnp.int32, 16) + (start + bb * 16)
                ok = (pos >= lo) & (pos < hi)
                rl_v[pl.ds(bb * 16, 16)] = jnp.where(
                    ok, rl_v[pl.ds(bb * 16, 16)], TRASHLOC)

            def _acc16(bb, _2):
                rv = rl_v[pl.ds(bb * 16, 16)]
                for t in range(16):
                    row = rv[t]
                    cnt_v[row, pl.ds(0, 16)] = cnt_v[row, pl.ds(0, 16)] + ones16
                return 0

            lax.fori_loop(0, K // 16, _acc16, 0)
            return 0

        lax.fori_loop(0, nch, _chunk, 0)

        @pl.when(g < NG)
        def _wb():
            for r in range(R_):
                dst_row = pl.multiple_of(r * N8 + g * C2, 8)
                pltpu.sync_copy(cnt_v.at[pl.ds(r * C2, C2)],
                                cnt.at[pl.ds(dst_row, C2)])


def _sc_count(rloc, bi):
    mesh = plsc.VectorSubcoreMesh(core_axis_name="c", subcore_axis_name="s")
    return pl.kernel(
        _sc_count_body,
        out_type=jax.ShapeDtypeStruct((R_ * N8, 16), jnp.float32),
        mesh=mesh,
        compiler_params=pltpu.CompilerParams(needs_layout_passes=False),
        scratch_types=[
            pltpu.VMEM((16,), jnp.int32),
            pltpu.VMEM((K,), jnp.int32),
            pltpu.VMEM((ACCR, 16), jnp.float32),
        ],
    )(rloc, bi)


def _tc_layer_body(hp_ref, agg_ref, cnt_ref, wroot_ref, wrel_ref, bias_ref,
                   out_ref):
    r = pl.program_id(1)

    @pl.when(r == 0)
    def _init():
        out_ref[...] = (
            jnp.dot(hp_ref[...], wroot_ref[...],
                    preferred_element_type=jnp.float32)
            + bias_ref[...])

    cnt = cnt_ref[:, 0:1]
    mean = agg_ref[...] * (1.0 / jnp.maximum(cnt, 1.0))
    out_ref[...] = out_ref[...] + jnp.dot(mean, wrel_ref[0],
                                          preferred_element_type=jnp.float32)

    @pl.when(r == R_ - 1)
    def _fin():
        out_ref[...] = jnp.maximum(out_ref[...], 0.0)


def _tc_layer(hp, agg, cnt, wroot, wrel, bias):
    nb = N_ // B
    ra = N8 // B
    return pl.pallas_call(
        _tc_layer_body,
        grid=(nb, R_),
        in_specs=[
            pl.BlockSpec((B, D_), lambda i, r: (i, 0)),
            pl.BlockSpec((B, D_), lambda i, r: (r * ra + i, 0)),
            pl.BlockSpec((B, 16), lambda i, r: (r * ra + i, 0)),
            pl.BlockSpec((D_, D_), lambda i, r: (0, 0)),
            pl.BlockSpec((1, D_, D_), lambda i, r: (r, 0, 0)),
            pl.BlockSpec((1, D_), lambda i, r: (0, 0)),
        ],
        out_specs=pl.BlockSpec((B, D_), lambda i, r: (i, 0)),
        out_shape=jax.ShapeDtypeStruct((N_, D_), jnp.float32),
    )(hp, agg, cnt, wroot, wrel, bias.reshape(1, D_))


def kernel(x, edge_index, edge_type, rel_w, root_w, bias):
    src = edge_index[0]
    dst = edge_index[1]

    order = jnp.argsort(dst * 8 + edge_type)
    src_s = src[order]
    dst_s = dst[order]
    et_s = edge_type[order]
    rowloc = et_s * C2 + (dst_s % C2)
    pad = E_PAD - E_
    srcs_p = jnp.concatenate([src_s, jnp.zeros((pad,), jnp.int32)])
    rloc_p = jnp.concatenate([rowloc, jnp.full((pad,), TRASHLOC, jnp.int32)])

    big = jnp.searchsorted(dst_s, jnp.arange(NG + 1, dtype=jnp.int32) * C2
                           ).astype(jnp.int32)
    gs = (jnp.arange(NW, dtype=jnp.int32)[:, None]
          + NW * jnp.arange(GPT, dtype=jnp.int32)[None, :])
    valid = gs < NG
    gl = jnp.where(valid, big[jnp.clip(gs, 0, NG - 1)], 0)
    gh = jnp.where(valid, big[jnp.clip(gs + 1, 0, NG)], 0)
    bi = jnp.concatenate([jnp.stack([gl, gh], axis=2).reshape(NW, 2 * GPT),
                          jnp.zeros((NW, 16 - 2 * GPT), jnp.int32)],
                         axis=1).reshape(-1)

    cnt = _sc_count(rloc_p, bi)
    hp = x
    for l in range(L_):
        agg = _sc_scatter(hp, srcs_p, rloc_p, bi)
        hp = _tc_layer(hp, agg, cnt, root_w[l], rel_w[l], bias[l])
    return hp
